# baseline (speedup 1.0000x reference)
import functools

import jax
import jax.numpy as jnp
from jax import lax
from jax.experimental import pallas as pl
from jax.experimental.pallas import tpu as pltpu
from jax.experimental.pallas import tpu_sc as plsc

H, W, D = 128, 128, 256
L = 16
DC = D // L
NUM_WORKERS = 32
H_PER_WORKER = H // NUM_WORKERS
PIECE = 64
HALVES = W // PIECE
NBUF = 4

_mesh = plsc.VectorSubcoreMesh(core_axis_name="c", subcore_axis_name="s")


@functools.partial(
    pl.kernel,
    mesh=_mesh,
    out_type=jax.ShapeDtypeStruct((H * W, D), jnp.float32),
    scratch_types=[
        pltpu.VMEM((W, D), jnp.float32),
        pltpu.VMEM((H_PER_WORKER, D), jnp.float32),
        pltpu.VMEM((NBUF, PIECE, D), jnp.float32),
        pltpu.SemaphoreType.DMA,
        pltpu.SemaphoreType.DMA,
        pltpu.SemaphoreType.DMA,
        pltpu.SemaphoreType.DMA,
        pltpu.SemaphoreType.DMA,
        pltpu.SemaphoreType.DMA,
    ],
)
def _pos_encoder(height_hbm, width_hbm, out_hbm,
                 width_v, hrows_v, ring, sem_w0, sem_w1,
                 sem0, sem1, sem2, sem3):
    wid = lax.axis_index("s") * 2 + lax.axis_index("c")
    base_h = wid * H_PER_WORKER

    w_half = (
        pltpu.async_copy(width_hbm.at[pl.ds(0, PIECE)],
                         width_v.at[pl.ds(0, PIECE)], sem_w0),
        pltpu.async_copy(width_hbm.at[pl.ds(PIECE, PIECE)],
                         width_v.at[pl.ds(PIECE, PIECE)], sem_w1),
    )
    pltpu.sync_copy(height_hbm.at[pl.ds(base_h, H_PER_WORKER)], hrows_v)

    sems = (sem0, sem1, sem2, sem3)
    pending = [None] * NBUF

    p = 0
    for half in range(HALVES):
        w_half[half].wait()
        for hh in range(H_PER_WORKER):
            slot = p % NBUF
            if pending[slot] is not None:
                pending[slot].wait()
            buf = ring.at[slot]
            hregs = tuple(hrows_v[hh, pl.ds(dc * L, L)] for dc in range(DC))

            @plsc.parallel_loop(0, PIECE, carry=hregs)
            def _body(w, carry, buf=buf, half=half):
                for dc in range(DC):
                    buf[w, pl.ds(dc * L, L)] = (
                        width_v[half * PIECE + w, pl.ds(dc * L, L)]
                        + carry[dc])
                return carry

            row0 = (base_h + hh) * W + half * PIECE
            pending[slot] = pltpu.async_copy(
                buf, out_hbm.at[pl.ds(row0, PIECE)], sems[slot])
            p += 1

    for cp in pending:
        cp.wait()


def kernel(height_table, width_table):
    return _pos_encoder(height_table, width_table)


# speedup vs baseline: 1.0320x; 1.0320x over previous
import functools

import jax
import jax.numpy as jnp
from jax import lax
from jax.experimental import pallas as pl
from jax.experimental.pallas import tpu as pltpu
from jax.experimental.pallas import tpu_sc as plsc

H, W, D = 128, 128, 256
L = 16
DC = D // L
NUM_WORKERS = 32
H_PER_WORKER = H // NUM_WORKERS

_mesh = plsc.VectorSubcoreMesh(core_axis_name="c", subcore_axis_name="s")


@functools.partial(
    pl.kernel,
    mesh=_mesh,
    out_type=jax.ShapeDtypeStruct((H * W, D), jnp.float32),
    scratch_types=[
        pltpu.VMEM((W, D), jnp.float32),
        pltpu.VMEM((H_PER_WORKER, D), jnp.float32),
        pltpu.VMEM((W, D), jnp.float32),
        pltpu.VMEM((W, D), jnp.float32),
        pltpu.SemaphoreType.DMA,
        pltpu.SemaphoreType.DMA,
    ],
)
def _pos_encoder(height_hbm, width_hbm, out_hbm,
                 width_v, hrows_v, buf0, buf1, sem0, sem1):
    wid = lax.axis_index("s") * 2 + lax.axis_index("c")
    base_h = wid * H_PER_WORKER

    pltpu.sync_copy(width_hbm, width_v)
    pltpu.sync_copy(height_hbm.at[pl.ds(base_h, H_PER_WORKER)], hrows_v)

    bufs = (buf0, buf1)
    sems = (sem0, sem1)
    pending = [None, None]

    for hh in range(H_PER_WORKER):
        slot = hh % 2
        buf = bufs[slot]
        if pending[slot] is not None:
            pending[slot].wait()

        hregs = tuple(hrows_v[hh, pl.ds(dc * L, L)] for dc in range(DC))

        @plsc.parallel_loop(0, W, carry=hregs)
        def _body(w, carry, buf=buf):
            for dc in range(DC):
                buf[w, pl.ds(dc * L, L)] = (
                    width_v[w, pl.ds(dc * L, L)] + carry[dc])
            return carry

        cp = pltpu.async_copy(
            buf, out_hbm.at[pl.ds((base_h + hh) * W, W)], sems[slot])
        pending[slot] = cp

    pending[0].wait()
    pending[1].wait()


def kernel(height_table, width_table):
    return _pos_encoder(height_table, width_table)


# speedup vs baseline: 1.0474x; 1.0149x over previous
import functools

import jax
import jax.numpy as jnp
from jax import lax
from jax.experimental import pallas as pl
from jax.experimental.pallas import tpu as pltpu
from jax.experimental.pallas import tpu_sc as plsc

H, W, D = 128, 128, 256
L = 16
DC = D // L
NUM_WORKERS = 32
H_PER_WORKER = H // NUM_WORKERS
HALF = W // 2

_mesh = plsc.VectorSubcoreMesh(core_axis_name="c", subcore_axis_name="s")


@functools.partial(
    pl.kernel,
    mesh=_mesh,
    out_type=jax.ShapeDtypeStruct((H * W, D), jnp.float32),
    scratch_types=[
        pltpu.VMEM((W, D), jnp.float32),
        pltpu.VMEM((H_PER_WORKER, D), jnp.float32),
        pltpu.VMEM((W, D), jnp.float32),
        pltpu.VMEM((W, D), jnp.float32),
        pltpu.SemaphoreType.DMA,
        pltpu.SemaphoreType.DMA,
        pltpu.SemaphoreType.DMA,
        pltpu.SemaphoreType.DMA,
    ],
)
def _pos_encoder(height_hbm, width_hbm, out_hbm,
                 width_v, hrows_v, buf0, buf1, sem_a, sem_b, sem_c, sem_d):
    wid = lax.axis_index("s") * 2 + lax.axis_index("c")
    base_h = wid * H_PER_WORKER

    cp_w0 = pltpu.async_copy(width_hbm.at[pl.ds(0, HALF)],
                             width_v.at[pl.ds(0, HALF)], sem_a)
    cp_w1 = pltpu.async_copy(width_hbm.at[pl.ds(HALF, HALF)],
                             width_v.at[pl.ds(HALF, HALF)], sem_b)
    pltpu.sync_copy(height_hbm.at[pl.ds(base_h, H_PER_WORKER)], hrows_v)

    def hreg_chunks(hh):
        return tuple(hrows_v[hh, pl.ds(dc * L, L)] for dc in range(DC))

    def compute(buf, hregs, lo, hi):
        @plsc.parallel_loop(lo, hi, carry=hregs)
        def _body(w, carry):
            for dc in range(DC):
                buf[w, pl.ds(dc * L, L)] = (
                    width_v[w, pl.ds(dc * L, L)] + carry[dc])
            return carry

    def out_rows(hh, off=0):
        return (base_h + hh) * W + off

    hregs = hreg_chunks(0)
    cp_w0.wait()
    compute(buf0, hregs, 0, HALF)
    cp_w1.wait()
    compute(buf0, hregs, HALF, W)
    dma0 = pltpu.async_copy(buf0, out_hbm.at[pl.ds(out_rows(0), W)], sem_c)

    compute(buf1, hreg_chunks(1), 0, W)
    dma1 = pltpu.async_copy(buf1, out_hbm.at[pl.ds(out_rows(1), W)], sem_d)

    dma0.wait()
    compute(buf0, hreg_chunks(2), 0, W)
    dma2 = pltpu.async_copy(buf0, out_hbm.at[pl.ds(out_rows(2), W)], sem_c)

    dma1.wait()
    hregs = hreg_chunks(3)
    compute(buf1, hregs, 0, HALF)
    dma3a = pltpu.async_copy(buf1.at[pl.ds(0, HALF)],
                             out_hbm.at[pl.ds(out_rows(3), HALF)], sem_a)
    compute(buf1, hregs, HALF, W)
    dma3b = pltpu.async_copy(buf1.at[pl.ds(HALF, HALF)],
                             out_hbm.at[pl.ds(out_rows(3, HALF), HALF)], sem_b)

    dma2.wait()
    dma3a.wait()
    dma3b.wait()


def kernel(height_table, width_table):
    return _pos_encoder(height_table, width_table)
